# Initial kernel scaffold; baseline (speedup 1.0000x reference)
import jax
import jax.numpy as jnp
from jax.experimental import pallas as pl


def kernel(dst_x, enc_w1, enc_b1, enc_w2, enc_b2, e2d_w, dec_w1, dec_b1, dec_w2, dec_b2, mask_nodes):
    raise NotImplementedError("write your pallas kernel here")


# speedup vs baseline: 5.7304x; 5.7304x over previous
import functools

import jax
import jax.numpy as jnp
from jax import lax
from jax.experimental import pallas as pl
from jax.experimental.pallas import tpu as pltpu
from jax.experimental.pallas import tpu_sc as plsc

N = 100000
D = 128
DH = 256
M = 50000

NC = 2
NS = 16
L = 16
NW = NC * NS
PER_W = 1568
M_PAD = NW * PER_W
CHUNKS = PER_W // L

BLK = 8192


def _tc_per_node_cost(x_ref, b1_ref, w2_ref, b2_ref, out_ref):
    v = jnp.maximum(b1_ref[...], 0.0) @ w2_ref[...] + b2_ref[...]
    vn = v / jnp.maximum(jnp.sqrt(jnp.sum(v * v)), 1e-8)
    x = x_ref[...]
    dot = jnp.sum(x * vn, axis=1)
    nrm = jnp.sqrt(jnp.sum(x * x, axis=1))
    r = 1.0 - dot / jnp.maximum(nrm, 1e-8)
    out_ref[...] = r * r


@functools.lru_cache(maxsize=1)
def _make_sc_gather_sum():
    mesh = plsc.VectorSubcoreMesh(
        core_axis_name="c", subcore_axis_name="s", num_cores=NC, num_subcores=NS
    )

    @functools.partial(
        pl.kernel,
        out_type=jax.ShapeDtypeStruct((NW, L), jnp.float32),
        mesh=mesh,
        compiler_params=pltpu.CompilerParams(needs_layout_passes=False),
        scratch_types=[
            pltpu.VMEM((N,), jnp.float32),
            pltpu.VMEM((PER_W,), jnp.int32),
            pltpu.VMEM((L,), jnp.float32),
        ],
    )
    def _sc_gather_sum(c_hbm, idx_hbm, out_hbm, c_v, idx_v, acc_v):
        wid = lax.axis_index("s") * NC + lax.axis_index("c")
        base = wid * PER_W
        pltpu.sync_copy(c_hbm, c_v)
        pltpu.sync_copy(idx_hbm.at[pl.ds(base, PER_W)], idx_v)

        def body(k, acc):
            idxs = idx_v[pl.ds(k * L, L)]
            vals = plsc.load_gather(c_v, [idxs])
            g = base + k * L + lax.iota(jnp.int32, L)
            return acc + jnp.where(g < M, vals, 0.0)

        acc = lax.fori_loop(0, CHUNKS, body, jnp.zeros((L,), jnp.float32))
        acc_v[...] = acc
        pltpu.sync_copy(acc_v, out_hbm.at[wid])

    return _sc_gather_sum


def kernel(dst_x, enc_w1, enc_b1, enc_w2, enc_b2, e2d_w,
           dec_w1, dec_b1, dec_w2, dec_b2, mask_nodes):
    c = pl.pallas_call(
        _tc_per_node_cost,
        grid=(pl.cdiv(N, BLK),),
        in_specs=[
            pl.BlockSpec((BLK, D), lambda i: (i, 0)),
            pl.BlockSpec((1, DH), lambda i: (0, 0)),
            pl.BlockSpec((DH, D), lambda i: (0, 0)),
            pl.BlockSpec((1, D), lambda i: (0, 0)),
        ],
        out_specs=pl.BlockSpec((BLK,), lambda i: (i,)),
        out_shape=jax.ShapeDtypeStruct((N,), jnp.float32),
    )(dst_x, dec_b1.reshape(1, DH), dec_w2, dec_b2.reshape(1, D))

    idx = jnp.zeros((M_PAD,), jnp.int32).at[:M].set(mask_nodes.astype(jnp.int32))
    partial = _make_sc_gather_sum()(c, idx)

    return jnp.sum(partial) / jnp.float32(M)


# speedup vs baseline: 5.9052x; 1.0305x over previous
import functools

import jax
import jax.numpy as jnp
from jax import lax
from jax.experimental import pallas as pl
from jax.experimental.pallas import tpu as pltpu
from jax.experimental.pallas import tpu_sc as plsc

N = 100000
D = 128
DH = 256
M = 50000

NC = 2
NS = 16
L = 16
NW = NC * NS
PER_W = 1568
M_PAD = NW * PER_W
CK = 112
NCK = PER_W // CK
GPC = CK // L

BLK = 8192


def _tc_dot_q(x_ref, b1_ref, w2_ref, b2_ref, dot_ref, q_ref):
    v = jnp.maximum(b1_ref[...], 0.0) @ w2_ref[...] + b2_ref[...]
    vn = v / jnp.maximum(jnp.sqrt(jnp.sum(v * v)), 1e-8)
    x = x_ref[...]
    dot_ref[...] = jnp.sum(x * vn, axis=1, keepdims=True)
    q_ref[...] = jnp.sum(x * x, axis=1, keepdims=True)


def _rsqrt16(x):
    i = plsc.bitcast(x, jnp.int32)
    y = plsc.bitcast(jnp.int32(0x5F3759DF) - (i >> 1), jnp.float32)
    for _ in range(3):
        y = y * (1.5 - 0.5 * x * y * y)
    return y


@functools.lru_cache(maxsize=1)
def _make_sc_gather_loss():
    mesh = plsc.VectorSubcoreMesh(
        core_axis_name="c", subcore_axis_name="s", num_cores=NC, num_subcores=NS
    )

    @functools.partial(
        pl.kernel,
        out_type=jax.ShapeDtypeStruct((NW, L), jnp.float32),
        mesh=mesh,
        compiler_params=pltpu.CompilerParams(needs_layout_passes=False),
        scratch_types=[
            pltpu.VMEM((NCK, CK), jnp.int32),
            pltpu.VMEM((2, CK), jnp.float32),
            pltpu.VMEM((2, CK), jnp.float32),
            pltpu.VMEM((L,), jnp.float32),
            pltpu.SemaphoreType.DMA,
            pltpu.SemaphoreType.DMA,
        ],
    )
    def _sc_gather_loss(dot_hbm, q_hbm, idx_hbm, out_hbm,
                        idx_v, dbuf, qbuf, acc_v, sem0, sem1):
        wid = lax.axis_index("s") * NC + lax.axis_index("c")
        base = wid * PER_W
        pltpu.sync_copy(idx_hbm.at[wid], idx_v)

        sems = (sem0, sem1)

        def fire(k):
            s = sems[k % 2]
            hd = pltpu.async_copy(dot_hbm.at[idx_v.at[k]], dbuf.at[k % 2], s)
            hq = pltpu.async_copy(q_hbm.at[idx_v.at[k]], qbuf.at[k % 2], s)
            return hd, hq

        acc = jnp.zeros((L,), jnp.float32)
        pending = fire(0)
        for k in range(NCK):
            nxt = fire(k + 1) if k + 1 < NCK else None
            pending[0].wait()
            pending[1].wait()
            for i in range(GPC):
                dot = dbuf[k % 2, pl.ds(i * L, L)]
                q = qbuf[k % 2, pl.ds(i * L, L)]
                s = q * _rsqrt16(jnp.maximum(q, 1e-30))
                r = 1.0 - dot / jnp.maximum(s, 1e-8)
                g = base + k * CK + i * L + lax.iota(jnp.int32, L)
                acc = acc + jnp.where(g < M, r * r, 0.0)
            pending = nxt

        acc_v[...] = acc
        pltpu.sync_copy(acc_v, out_hbm.at[wid])

    return _sc_gather_loss


def kernel(dst_x, enc_w1, enc_b1, enc_w2, enc_b2, e2d_w,
           dec_w1, dec_b1, dec_w2, dec_b2, mask_nodes):
    dot, q = pl.pallas_call(
        _tc_dot_q,
        grid=(pl.cdiv(N, BLK),),
        in_specs=[
            pl.BlockSpec((BLK, D), lambda i: (i, 0)),
            pl.BlockSpec((1, DH), lambda i: (0, 0)),
            pl.BlockSpec((DH, D), lambda i: (0, 0)),
            pl.BlockSpec((1, D), lambda i: (0, 0)),
        ],
        out_specs=[
            pl.BlockSpec((BLK, 1), lambda i: (i, 0)),
            pl.BlockSpec((BLK, 1), lambda i: (i, 0)),
        ],
        out_shape=[
            jax.ShapeDtypeStruct((N, 1), jnp.float32),
            jax.ShapeDtypeStruct((N, 1), jnp.float32),
        ],
    )(dst_x, dec_b1.reshape(1, DH), dec_w2, dec_b2.reshape(1, D))

    idx = jnp.zeros((M_PAD,), jnp.int32).at[:M].set(mask_nodes.astype(jnp.int32))
    partial = _make_sc_gather_loss()(
        dot.reshape(N), q.reshape(N), idx.reshape(NW, NCK, CK))

    return jnp.sum(partial) / jnp.float32(M)


# speedup vs baseline: 12.0905x; 2.0474x over previous
import functools

import jax
import jax.numpy as jnp
from jax import lax
from jax.experimental import pallas as pl
from jax.experimental.pallas import tpu as pltpu
from jax.experimental.pallas import tpu_sc as plsc

N = 100000
D = 128
DH = 256
M = 50000

NC = 2
NS = 16
L = 16
NW = NC * NS
PER_W = 1568
M_PAD = NW * PER_W
CK = 112
NCK = PER_W // CK
GPC = CK // L

BLK = 8192


def _tc_dot_q(x_ref, b1_ref, w2_ref, b2_ref, dot_ref, q_ref):
    v = jnp.maximum(b1_ref[...], 0.0) @ w2_ref[...] + b2_ref[...]
    vn = v / jnp.maximum(jnp.sqrt(jnp.sum(v * v)), 1e-8)
    x = x_ref[...]
    contract = (((1,), (1,)), ((), ()))
    dot = lax.dot_general(vn, x, contract,
                          preferred_element_type=jnp.float32)
    ones = jnp.ones((1, D), jnp.float32)
    q = lax.dot_general(ones, x * x, contract,
                        preferred_element_type=jnp.float32)
    dot_ref[...] = dot.reshape(BLK)
    q_ref[...] = q.reshape(BLK)


def _rsqrt16(x):
    i = plsc.bitcast(x, jnp.int32)
    y = plsc.bitcast(jnp.int32(0x5F3759DF) - (i >> 1), jnp.float32)
    for _ in range(3):
        y = y * (1.5 - 0.5 * x * y * y)
    return y


@functools.lru_cache(maxsize=1)
def _make_sc_gather_loss():
    mesh = plsc.VectorSubcoreMesh(
        core_axis_name="c", subcore_axis_name="s", num_cores=NC, num_subcores=NS
    )

    @functools.partial(
        pl.kernel,
        out_type=jax.ShapeDtypeStruct((NW, L), jnp.float32),
        mesh=mesh,
        compiler_params=pltpu.CompilerParams(needs_layout_passes=False),
        scratch_types=[
            pltpu.VMEM((NCK, CK), jnp.int32),
            pltpu.VMEM((2, CK), jnp.float32),
            pltpu.VMEM((2, CK), jnp.float32),
            pltpu.VMEM((L,), jnp.float32),
            pltpu.SemaphoreType.DMA,
            pltpu.SemaphoreType.DMA,
        ],
    )
    def _sc_gather_loss(dot_hbm, q_hbm, idx_hbm, out_hbm,
                        idx_v, dbuf, qbuf, acc_v, sem0, sem1):
        wid = lax.axis_index("s") * NC + lax.axis_index("c")
        base = wid * PER_W
        pltpu.sync_copy(idx_hbm.at[wid], idx_v)

        sems = (sem0, sem1)

        def fire(k):
            s = sems[k % 2]
            hd = pltpu.async_copy(dot_hbm.at[idx_v.at[k]], dbuf.at[k % 2], s)
            hq = pltpu.async_copy(q_hbm.at[idx_v.at[k]], qbuf.at[k % 2], s)
            return hd, hq

        acc = jnp.zeros((L,), jnp.float32)
        pending = fire(0)
        for k in range(NCK):
            nxt = fire(k + 1) if k + 1 < NCK else None
            pending[0].wait()
            pending[1].wait()
            for i in range(GPC):
                dot = dbuf[k % 2, pl.ds(i * L, L)]
                q = qbuf[k % 2, pl.ds(i * L, L)]
                s = q * _rsqrt16(jnp.maximum(q, 1e-30))
                r = 1.0 - dot / jnp.maximum(s, 1e-8)
                g = base + k * CK + i * L + lax.iota(jnp.int32, L)
                acc = acc + jnp.where(g < M, r * r, 0.0)
            pending = nxt

        acc_v[...] = acc
        pltpu.sync_copy(acc_v, out_hbm.at[wid])

    return _sc_gather_loss


def kernel(dst_x, enc_w1, enc_b1, enc_w2, enc_b2, e2d_w,
           dec_w1, dec_b1, dec_w2, dec_b2, mask_nodes):
    dot, q = pl.pallas_call(
        _tc_dot_q,
        grid=(pl.cdiv(N, BLK),),
        in_specs=[
            pl.BlockSpec((BLK, D), lambda i: (i, 0)),
            pl.BlockSpec((1, DH), lambda i: (0, 0)),
            pl.BlockSpec((DH, D), lambda i: (0, 0)),
            pl.BlockSpec((1, D), lambda i: (0, 0)),
        ],
        out_specs=[
            pl.BlockSpec((BLK,), lambda i: (i,)),
            pl.BlockSpec((BLK,), lambda i: (i,)),
        ],
        out_shape=[
            jax.ShapeDtypeStruct((N,), jnp.float32),
            jax.ShapeDtypeStruct((N,), jnp.float32),
        ],
    )(dst_x, dec_b1.reshape(1, DH), dec_w2, dec_b2.reshape(1, D))

    idx = jnp.zeros((M_PAD,), jnp.int32).at[:M].set(mask_nodes.astype(jnp.int32))
    partial = _make_sc_gather_loss()(dot, q, idx.reshape(NW, NCK, CK))

    return jnp.sum(partial) / jnp.float32(M)


# speedup vs baseline: 12.6045x; 1.0425x over previous
import functools

import jax
import jax.numpy as jnp
from jax import lax
from jax.experimental import pallas as pl
from jax.experimental.pallas import tpu as pltpu
from jax.experimental.pallas import tpu_sc as plsc

N = 100000
D = 128
DH = 256
M = 50000

NC = 2
NS = 16
L = 16
NW = NC * NS
PER_W = 1568
M_PAD = NW * PER_W
CK = 112
NCK = PER_W // CK
GPC = CK // L

BLK = 8192


def _tc_dot_q(xa_ref, xb_ref, b1_ref, w2_ref, b2_ref, dot_ref, q_ref):
    v = jnp.maximum(b1_ref[...], 0.0) @ w2_ref[...] + b2_ref[...]
    vn = v / jnp.maximum(jnp.sqrt(jnp.sum(v * v)), 1e-8)
    xa = xa_ref[...]
    xb = xb_ref[...]
    contract = (((1,), (1,)), ((), ()))
    ones = jnp.ones((1, D), jnp.float32)

    def dq(x):
        dot = lax.dot_general(vn, x, contract,
                              preferred_element_type=jnp.float32)
        q = lax.dot_general(ones, x * x, contract,
                            preferred_element_type=jnp.float32)
        return dot.reshape(BLK), q.reshape(BLK)

    da, qa = dq(xa)
    db, qb = dq(xb)
    dot_ref[...] = jnp.concatenate([da, db])
    q_ref[...] = jnp.concatenate([qa, qb])


def _rsqrt16(x):
    i = plsc.bitcast(x, jnp.int32)
    y = plsc.bitcast(jnp.int32(0x5F3759DF) - (i >> 1), jnp.float32)
    for _ in range(3):
        y = y * (1.5 - 0.5 * x * y * y)
    return y


@functools.lru_cache(maxsize=1)
def _make_sc_gather_loss():
    mesh = plsc.VectorSubcoreMesh(
        core_axis_name="c", subcore_axis_name="s", num_cores=NC, num_subcores=NS
    )

    @functools.partial(
        pl.kernel,
        out_type=jax.ShapeDtypeStruct((NW, L), jnp.float32),
        mesh=mesh,
        compiler_params=pltpu.CompilerParams(needs_layout_passes=False),
        scratch_types=[
            pltpu.VMEM((NCK, CK), jnp.int32),
            pltpu.VMEM((2, CK), jnp.float32),
            pltpu.VMEM((2, CK), jnp.float32),
            pltpu.VMEM((L,), jnp.float32),
            pltpu.SemaphoreType.DMA,
            pltpu.SemaphoreType.DMA,
        ],
    )
    def _sc_gather_loss(dot_hbm, q_hbm, idx_hbm, out_hbm,
                        idx_v, dbuf, qbuf, acc_v, sem0, sem1):
        wid = lax.axis_index("s") * NC + lax.axis_index("c")
        base = wid * PER_W
        pltpu.sync_copy(idx_hbm.at[wid], idx_v)

        sems = (sem0, sem1)

        def fire(k):
            s = sems[k % 2]
            hd = pltpu.async_copy(dot_hbm.at[idx_v.at[k]], dbuf.at[k % 2], s)
            hq = pltpu.async_copy(q_hbm.at[idx_v.at[k]], qbuf.at[k % 2], s)
            return hd, hq

        acc = jnp.zeros((L,), jnp.float32)
        pending = fire(0)
        for k in range(NCK):
            nxt = fire(k + 1) if k + 1 < NCK else None
            pending[0].wait()
            pending[1].wait()
            for i in range(GPC):
                dot = dbuf[k % 2, pl.ds(i * L, L)]
                q = qbuf[k % 2, pl.ds(i * L, L)]
                s = q * _rsqrt16(jnp.maximum(q, 1e-30))
                r = 1.0 - dot / jnp.maximum(s, 1e-8)
                g = base + k * CK + i * L + lax.iota(jnp.int32, L)
                acc = acc + jnp.where(g < M, r * r, 0.0)
            pending = nxt

        acc_v[...] = acc
        pltpu.sync_copy(acc_v, out_hbm.at[wid])

    return _sc_gather_loss


def kernel(dst_x, enc_w1, enc_b1, enc_w2, enc_b2, e2d_w,
           dec_w1, dec_b1, dec_w2, dec_b2, mask_nodes):
    dot, q = pl.pallas_call(
        _tc_dot_q,
        grid=(pl.cdiv(N, 2 * BLK),),
        in_specs=[
            pl.BlockSpec((BLK, D), lambda i: (2 * i, 0)),
            pl.BlockSpec((BLK, D),
                         lambda i: (jnp.minimum(2 * i + 1, N // BLK), 0)),
            pl.BlockSpec((1, DH), lambda i: (0, 0)),
            pl.BlockSpec((DH, D), lambda i: (0, 0)),
            pl.BlockSpec((1, D), lambda i: (0, 0)),
        ],
        out_specs=[
            pl.BlockSpec((2 * BLK,), lambda i: (i,)),
            pl.BlockSpec((2 * BLK,), lambda i: (i,)),
        ],
        out_shape=[
            jax.ShapeDtypeStruct((N,), jnp.float32),
            jax.ShapeDtypeStruct((N,), jnp.float32),
        ],
    )(dst_x, dst_x, dec_b1.reshape(1, DH), dec_w2, dec_b2.reshape(1, D))

    idx = jnp.zeros((M_PAD,), jnp.int32).at[:M].set(mask_nodes.astype(jnp.int32))
    partial = _make_sc_gather_loss()(dot, q, idx.reshape(NW, NCK, CK))

    return jnp.sum(partial) / jnp.float32(M)


# speedup vs baseline: 13.0837x; 1.0380x over previous
import functools

import jax
import jax.numpy as jnp
from jax import lax
from jax.experimental import pallas as pl
from jax.experimental.pallas import tpu as pltpu
from jax.experimental.pallas import tpu_sc as plsc

N = 100000
D = 128
DH = 256
M = 50000

NC = 2
NS = 16
L = 16
NW = NC * NS
PER_W = 1568
M_PAD = NW * PER_W
CK = 112
NCK = PER_W // CK
GPC = CK // L

BLK = 8192


def _tc_dot_q(xa_ref, xb_ref, b1_ref, w2_ref, b2_ref, dot_ref, q_ref):
    v = jnp.maximum(b1_ref[...], 0.0) @ w2_ref[...] + b2_ref[...]
    vn = v / jnp.maximum(jnp.sqrt(jnp.sum(v * v)), 1e-8)
    xa = xa_ref[...]
    xb = xb_ref[...]
    contract = (((1,), (1,)), ((), ()))
    ones = jnp.ones((1, D), jnp.float32)

    def dq(x):
        dot = lax.dot_general(vn, x, contract,
                              preferred_element_type=jnp.float32)
        q = lax.dot_general(ones, x * x, contract,
                            preferred_element_type=jnp.float32)
        return dot.reshape(BLK), q.reshape(BLK)

    da, qa = dq(xa)
    db, qb = dq(xb)
    dot_ref[...] = jnp.concatenate([da, db])
    q_ref[...] = jnp.concatenate([qa, qb])


def _rsqrt16(x):
    i = plsc.bitcast(x, jnp.int32)
    y = plsc.bitcast(jnp.int32(0x5F3759DF) - (i >> 1), jnp.float32)
    for _ in range(3):
        y = y * (1.5 - 0.5 * x * y * y)
    return y


@functools.lru_cache(maxsize=1)
def _make_sc_gather_loss():
    mesh = plsc.VectorSubcoreMesh(
        core_axis_name="c", subcore_axis_name="s", num_cores=NC, num_subcores=NS
    )

    @functools.partial(
        pl.kernel,
        out_type=jax.ShapeDtypeStruct((NW, L), jnp.float32),
        mesh=mesh,
        compiler_params=pltpu.CompilerParams(needs_layout_passes=False),
        scratch_types=[
            pltpu.VMEM((NCK, CK), jnp.int32),
            pltpu.VMEM((4, CK), jnp.float32),
            pltpu.VMEM((4, CK), jnp.float32),
            pltpu.VMEM((L,), jnp.float32),
            pltpu.SemaphoreType.DMA,
            pltpu.SemaphoreType.DMA,
            pltpu.SemaphoreType.DMA,
            pltpu.SemaphoreType.DMA,
        ],
    )
    def _sc_gather_loss(dot_hbm, q_hbm, idx_hbm, out_hbm,
                        idx_v, dbuf, qbuf, acc_v, sem0, sem1, sem2, sem3):
        wid = lax.axis_index("s") * NC + lax.axis_index("c")
        base = wid * PER_W
        pltpu.sync_copy(idx_hbm.at[wid], idx_v)

        sems = (sem0, sem1, sem2, sem3)
        RB = 4

        def fire(k):
            s = sems[k % RB]
            hd = pltpu.async_copy(dot_hbm.at[idx_v.at[k]], dbuf.at[k % RB], s)
            hq = pltpu.async_copy(q_hbm.at[idx_v.at[k]], qbuf.at[k % RB], s)
            return hd, hq

        acc = jnp.zeros((L,), jnp.float32)
        pend = [fire(k) for k in range(RB - 1)]
        for k in range(NCK):
            if k + RB - 1 < NCK:
                pend.append(fire(k + RB - 1))
            hd, hq = pend.pop(0)
            hd.wait()
            hq.wait()
            for i in range(GPC):
                dot = dbuf[k % RB, pl.ds(i * L, L)]
                q = qbuf[k % RB, pl.ds(i * L, L)]
                s = q * _rsqrt16(jnp.maximum(q, 1e-30))
                r = 1.0 - dot / jnp.maximum(s, 1e-8)
                g = base + k * CK + i * L + lax.iota(jnp.int32, L)
                acc = acc + jnp.where(g < M, r * r, 0.0)

        acc_v[...] = acc
        pltpu.sync_copy(acc_v, out_hbm.at[wid])

    return _sc_gather_loss


def kernel(dst_x, enc_w1, enc_b1, enc_w2, enc_b2, e2d_w,
           dec_w1, dec_b1, dec_w2, dec_b2, mask_nodes):
    dot, q = pl.pallas_call(
        _tc_dot_q,
        grid=(pl.cdiv(N, 2 * BLK),),
        in_specs=[
            pl.BlockSpec((BLK, D), lambda i: (2 * i, 0)),
            pl.BlockSpec((BLK, D),
                         lambda i: (jnp.minimum(2 * i + 1, N // BLK), 0)),
            pl.BlockSpec((1, DH), lambda i: (0, 0)),
            pl.BlockSpec((DH, D), lambda i: (0, 0)),
            pl.BlockSpec((1, D), lambda i: (0, 0)),
        ],
        out_specs=[
            pl.BlockSpec((2 * BLK,), lambda i: (i,)),
            pl.BlockSpec((2 * BLK,), lambda i: (i,)),
        ],
        out_shape=[
            jax.ShapeDtypeStruct((N,), jnp.float32),
            jax.ShapeDtypeStruct((N,), jnp.float32),
        ],
    )(dst_x, dst_x, dec_b1.reshape(1, DH), dec_w2, dec_b2.reshape(1, D))

    idx = jnp.zeros((M_PAD,), jnp.int32).at[:M].set(mask_nodes.astype(jnp.int32))
    partial = _make_sc_gather_loss()(dot, q, idx.reshape(NW, NCK, CK))

    return jnp.sum(partial) / jnp.float32(M)
